# baseline (speedup 1.0000x reference)
import jax
import jax.numpy as jnp
import numpy as np
from jax import lax
from jax.experimental import pallas as pl
from jax.experimental.pallas import tpu as pltpu
from jax.experimental.pallas import tpu_sc as plsc

N_ATOMS = 10000
N_BONDS = 320000
D = 128

NC = 2
NS = 16
NW = NC * NS

CH = 128
BONDS_PAD = 327680
PAD = BONDS_PAD - N_BONDS
DUMMY = 64
ACC_ROWS = N_ATOMS + DUMMY

MLP_BLOCK = 16384
GROUPS0 = 48
GROUPS1 = 32
CHUNK0_ROWS = GROUPS0 * CH * NW
CHUNK1_ROWS = GROUPS1 * CH * NW

INIT_TILES = 10
INIT_ROWS = N_ATOMS // INIT_TILES
COMBINE_BLOCK = 1000

_PAD_IDX = np.int32(N_ATOMS) + np.arange(PAD, dtype=np.int32) % np.int32(DUMMY)


def _mlp_body(x_ref, wm_ref, bm_ref, wg_ref, bg_ref, o_ref):
    x = x_ref[...].astype(jnp.bfloat16)
    zm = jnp.dot(x, wm_ref[...].astype(jnp.bfloat16),
                 preferred_element_type=jnp.float32) + bm_ref[...]
    zg = jnp.dot(x, wg_ref[...].astype(jnp.bfloat16),
                 preferred_element_type=jnp.float32) + bg_ref[...]
    o_ref[...] = zm * jax.nn.sigmoid(zm) * jax.nn.sigmoid(zg)


def _mlp_chunk(block_off, rows, bond_features, W_main, b_main, W_gate, b_gate):
    return pl.pallas_call(
        _mlp_body,
        grid=(rows // MLP_BLOCK,),
        in_specs=[
            pl.BlockSpec((MLP_BLOCK, D), lambda i: (i + block_off, 0)),
            pl.BlockSpec((D, D), lambda i: (0, 0)),
            pl.BlockSpec((1, D), lambda i: (0, 0)),
            pl.BlockSpec((D, D), lambda i: (0, 0)),
            pl.BlockSpec((1, D), lambda i: (0, 0)),
        ],
        out_specs=pl.BlockSpec((MLP_BLOCK, D), lambda i: (i, 0)),
        out_shape=jax.ShapeDtypeStruct((rows, D), jnp.float32),
    )(bond_features, W_main, b_main.reshape(1, D), W_gate, b_gate.reshape(1, D))


def _make_sc_scatter(first, groups, dst_row_base):
    def body(msg_hbm, dst_hbm, init_hbm, out_hbm, acc_sh, idx_v, buf_v,
             sem0, sem1):
        c = lax.axis_index("c")
        s = lax.axis_index("s")
        w = s * NC + c
        base = w * groups * CH

        @pl.when(s < INIT_TILES)
        def _init():
            if first:
                src = init_hbm.at[pl.ds(s * INIT_ROWS, INIT_ROWS)]
            else:
                src = init_hbm.at[c, pl.ds(s * INIT_ROWS, INIT_ROWS)]
            pltpu.sync_copy(src, acc_sh.at[pl.ds(s * INIT_ROWS, INIT_ROWS)])

        pltpu.sync_copy(dst_hbm.at[pl.ds(dst_row_base + w * groups, groups)],
                        idx_v)
        plsc.subcore_barrier()

        sems = (sem0, sem1)
        pltpu.async_copy(msg_hbm.at[pl.ds(base, CH)], buf_v.at[0], sems[0])

        def pair(k, carry):
            for b in range(2):
                g = 2 * k + b
                pltpu.make_async_copy(msg_hbm.at[pl.ds(base + g * CH, CH)],
                                      buf_v.at[b], sems[b]).wait()

                @pl.when(g + 1 < groups)
                def _next():
                    pltpu.async_copy(msg_hbm.at[pl.ds(base + (g + 1) * CH, CH)],
                                     buf_v.at[1 - b], sems[1 - b])

                pltpu.sync_copy(buf_v.at[b], acc_sh.at[idx_v.at[g]], add=True)
            return carry

        lax.fori_loop(0, groups // 2, pair, 0)
        plsc.subcore_barrier()

        @pl.when(s < INIT_TILES)
        def _out():
            pltpu.sync_copy(acc_sh.at[pl.ds(s * INIT_ROWS, INIT_ROWS)],
                            out_hbm.at[c, pl.ds(s * INIT_ROWS, INIT_ROWS)])

    return pl.kernel(
        body,
        mesh=plsc.VectorSubcoreMesh(core_axis_name="c", subcore_axis_name="s"),
        out_type=jax.ShapeDtypeStruct((NC, N_ATOMS, D), jnp.float32),
        scratch_types=[
            pltpu.VMEM_SHARED((ACC_ROWS, D), jnp.float32),
            pltpu.VMEM((groups, CH), jnp.int32),
            pltpu.VMEM((2, CH, D), jnp.float32),
            pltpu.SemaphoreType.DMA,
            pltpu.SemaphoreType.DMA,
        ],
    )


def _combine_body(p_ref, a_ref, o_ref):
    o_ref[...] = p_ref[0] + p_ref[1] - a_ref[...]


def _combine(partials, atom_features):
    return pl.pallas_call(
        _combine_body,
        grid=(N_ATOMS // COMBINE_BLOCK,),
        in_specs=[
            pl.BlockSpec((NC, COMBINE_BLOCK, D), lambda i: (0, i, 0)),
            pl.BlockSpec((COMBINE_BLOCK, D), lambda i: (i, 0)),
        ],
        out_specs=pl.BlockSpec((COMBINE_BLOCK, D), lambda i: (i, 0)),
        out_shape=jax.ShapeDtypeStruct((N_ATOMS, D), jnp.float32),
    )(partials, atom_features)


def kernel(atom_features, bond_features, bond_atom_indices, W_main, b_main, W_gate, b_gate):
    dst = bond_atom_indices[:, 1]
    dst_pad = jnp.concatenate([dst, jnp.asarray(_PAD_IDX)]).reshape(
        BONDS_PAD // CH, CH)
    msg0 = _mlp_chunk(0, CHUNK0_ROWS, bond_features,
                      W_main, b_main, W_gate, b_gate)
    msg1 = _mlp_chunk(CHUNK0_ROWS // MLP_BLOCK, CHUNK1_ROWS, bond_features,
                      W_main, b_main, W_gate, b_gate)
    p0 = _make_sc_scatter(True, GROUPS0, 0)(msg0, dst_pad, atom_features)
    p1 = _make_sc_scatter(False, GROUPS1, CHUNK0_ROWS // CH)(msg1, dst_pad, p0)
    return _combine(p1, atom_features)


# speedup vs baseline: 1.0140x; 1.0140x over previous
import jax
import jax.numpy as jnp
import numpy as np
from jax import lax
from jax.experimental import pallas as pl
from jax.experimental.pallas import tpu as pltpu
from jax.experimental.pallas import tpu_sc as plsc

N_ATOMS = 10000
N_BONDS = 320000
D = 128

NC = 2
NS = 16
NW = NC * NS

CH = 128
BONDS_PAD = 327680
PAD = BONDS_PAD - N_BONDS
DUMMY = 64
ACC_ROWS = N_ATOMS + DUMMY

MLP_BLOCK = 16384
GROUPS0 = 40
GROUPS1 = 40
CHUNK0_ROWS = GROUPS0 * CH * NW
CHUNK1_ROWS = GROUPS1 * CH * NW

INIT_TILES = 10
INIT_ROWS = N_ATOMS // INIT_TILES
COMBINE_BLOCK = 1000

_PAD_IDX = np.int32(N_ATOMS) + np.arange(PAD, dtype=np.int32) % np.int32(DUMMY)


def _mlp_body(x_ref, wm_ref, bm_ref, wg_ref, bg_ref, o_ref):
    x = x_ref[...].astype(jnp.bfloat16)
    zm = jnp.dot(x, wm_ref[...].astype(jnp.bfloat16),
                 preferred_element_type=jnp.float32) + bm_ref[...]
    zg = jnp.dot(x, wg_ref[...].astype(jnp.bfloat16),
                 preferred_element_type=jnp.float32) + bg_ref[...]
    o_ref[...] = zm * jax.nn.sigmoid(zm) * jax.nn.sigmoid(zg)


def _mlp_chunk(block_off, rows, bond_features, W_main, b_main, W_gate, b_gate):
    return pl.pallas_call(
        _mlp_body,
        grid=(rows // MLP_BLOCK,),
        in_specs=[
            pl.BlockSpec((MLP_BLOCK, D), lambda i: (i + block_off, 0)),
            pl.BlockSpec((D, D), lambda i: (0, 0)),
            pl.BlockSpec((1, D), lambda i: (0, 0)),
            pl.BlockSpec((D, D), lambda i: (0, 0)),
            pl.BlockSpec((1, D), lambda i: (0, 0)),
        ],
        out_specs=pl.BlockSpec((MLP_BLOCK, D), lambda i: (i, 0)),
        out_shape=jax.ShapeDtypeStruct((rows, D), jnp.float32),
    )(bond_features, W_main, b_main.reshape(1, D), W_gate, b_gate.reshape(1, D))


def _make_sc_scatter(first, groups, dst_row_base):
    def body(msg_hbm, dst_hbm, init_hbm, out_hbm, acc_sh, idx_v, buf_v,
             sem0, sem1):
        c = lax.axis_index("c")
        s = lax.axis_index("s")
        w = s * NC + c
        base = w * groups * CH

        @pl.when(s < INIT_TILES)
        def _init():
            if first:
                src = init_hbm.at[pl.ds(s * INIT_ROWS, INIT_ROWS)]
            else:
                src = init_hbm.at[c, pl.ds(s * INIT_ROWS, INIT_ROWS)]
            pltpu.sync_copy(src, acc_sh.at[pl.ds(s * INIT_ROWS, INIT_ROWS)])

        pltpu.sync_copy(dst_hbm.at[pl.ds(dst_row_base + w * groups, groups)],
                        idx_v)
        plsc.subcore_barrier()

        sems = (sem0, sem1)
        pltpu.async_copy(msg_hbm.at[pl.ds(base, CH)], buf_v.at[0], sems[0])

        def pair(k, carry):
            for b in range(2):
                g = 2 * k + b
                pltpu.make_async_copy(msg_hbm.at[pl.ds(base + g * CH, CH)],
                                      buf_v.at[b], sems[b]).wait()

                @pl.when(g + 1 < groups)
                def _next():
                    pltpu.async_copy(msg_hbm.at[pl.ds(base + (g + 1) * CH, CH)],
                                     buf_v.at[1 - b], sems[1 - b])

                pltpu.sync_copy(buf_v.at[b], acc_sh.at[idx_v.at[g]], add=True)
            return carry

        lax.fori_loop(0, groups // 2, pair, 0)
        plsc.subcore_barrier()

        @pl.when(s < INIT_TILES)
        def _out():
            pltpu.sync_copy(acc_sh.at[pl.ds(s * INIT_ROWS, INIT_ROWS)],
                            out_hbm.at[c, pl.ds(s * INIT_ROWS, INIT_ROWS)])

    return pl.kernel(
        body,
        mesh=plsc.VectorSubcoreMesh(core_axis_name="c", subcore_axis_name="s"),
        out_type=jax.ShapeDtypeStruct((NC, N_ATOMS, D), jnp.float32),
        scratch_types=[
            pltpu.VMEM_SHARED((ACC_ROWS, D), jnp.float32),
            pltpu.VMEM((groups, CH), jnp.int32),
            pltpu.VMEM((2, CH, D), jnp.float32),
            pltpu.SemaphoreType.DMA,
            pltpu.SemaphoreType.DMA,
        ],
    )


def _combine_body(p_ref, a_ref, o_ref):
    o_ref[...] = p_ref[0] + p_ref[1] - a_ref[...]


def _combine(partials, atom_features):
    return pl.pallas_call(
        _combine_body,
        grid=(N_ATOMS // COMBINE_BLOCK,),
        in_specs=[
            pl.BlockSpec((NC, COMBINE_BLOCK, D), lambda i: (0, i, 0)),
            pl.BlockSpec((COMBINE_BLOCK, D), lambda i: (i, 0)),
        ],
        out_specs=pl.BlockSpec((COMBINE_BLOCK, D), lambda i: (i, 0)),
        out_shape=jax.ShapeDtypeStruct((N_ATOMS, D), jnp.float32),
    )(partials, atom_features)


def kernel(atom_features, bond_features, bond_atom_indices, W_main, b_main, W_gate, b_gate):
    dst = bond_atom_indices[:, 1]
    dst_pad = jnp.concatenate([dst, jnp.asarray(_PAD_IDX)]).reshape(
        BONDS_PAD // CH, CH)
    msg0 = _mlp_chunk(0, CHUNK0_ROWS, bond_features,
                      W_main, b_main, W_gate, b_gate)
    msg1 = _mlp_chunk(CHUNK0_ROWS // MLP_BLOCK, CHUNK1_ROWS, bond_features,
                      W_main, b_main, W_gate, b_gate)
    p0 = _make_sc_scatter(True, GROUPS0, 0)(msg0, dst_pad, atom_features)
    p1 = _make_sc_scatter(False, GROUPS1, CHUNK0_ROWS // CH)(msg1, dst_pad, p0)
    return _combine(p1, atom_features)


# speedup vs baseline: 1.0315x; 1.0173x over previous
import jax
import jax.numpy as jnp
import numpy as np
from jax import lax
from jax.experimental import pallas as pl
from jax.experimental.pallas import tpu as pltpu
from jax.experimental.pallas import tpu_sc as plsc

N_ATOMS = 10000
N_BONDS = 320000
D = 128

NC = 2
NS = 16
NW = NC * NS

CH = 128
BONDS_PAD = 327680
PAD = BONDS_PAD - N_BONDS
DUMMY = 64
ACC_ROWS = N_ATOMS + DUMMY

MLP_BLOCK = 16384
GROUPS0 = 32
GROUPS1 = 48
CHUNK0_ROWS = GROUPS0 * CH * NW
CHUNK1_ROWS = GROUPS1 * CH * NW

INIT_TILES = 10
INIT_ROWS = N_ATOMS // INIT_TILES
COMBINE_BLOCK = 1000

_PAD_IDX = np.int32(N_ATOMS) + np.arange(PAD, dtype=np.int32) % np.int32(DUMMY)


def _mlp_body(x_ref, wm_ref, bm_ref, wg_ref, bg_ref, o_ref):
    x = x_ref[...].astype(jnp.bfloat16)
    zm = jnp.dot(x, wm_ref[...].astype(jnp.bfloat16),
                 preferred_element_type=jnp.float32) + bm_ref[...]
    zg = jnp.dot(x, wg_ref[...].astype(jnp.bfloat16),
                 preferred_element_type=jnp.float32) + bg_ref[...]
    o_ref[...] = zm * jax.nn.sigmoid(zm) * jax.nn.sigmoid(zg)


def _mlp_chunk(block_off, rows, bond_features, W_main, b_main, W_gate, b_gate):
    return pl.pallas_call(
        _mlp_body,
        grid=(rows // MLP_BLOCK,),
        in_specs=[
            pl.BlockSpec((MLP_BLOCK, D), lambda i: (i + block_off, 0)),
            pl.BlockSpec((D, D), lambda i: (0, 0)),
            pl.BlockSpec((1, D), lambda i: (0, 0)),
            pl.BlockSpec((D, D), lambda i: (0, 0)),
            pl.BlockSpec((1, D), lambda i: (0, 0)),
        ],
        out_specs=pl.BlockSpec((MLP_BLOCK, D), lambda i: (i, 0)),
        out_shape=jax.ShapeDtypeStruct((rows, D), jnp.float32),
    )(bond_features, W_main, b_main.reshape(1, D), W_gate, b_gate.reshape(1, D))


def _make_sc_scatter(first, groups, dst_row_base):
    def body(msg_hbm, dst_hbm, init_hbm, out_hbm, acc_sh, idx_v, buf_v,
             sem0, sem1):
        c = lax.axis_index("c")
        s = lax.axis_index("s")
        w = s * NC + c
        base = w * groups * CH

        @pl.when(s < INIT_TILES)
        def _init():
            if first:
                src = init_hbm.at[pl.ds(s * INIT_ROWS, INIT_ROWS)]
            else:
                src = init_hbm.at[c, pl.ds(s * INIT_ROWS, INIT_ROWS)]
            pltpu.sync_copy(src, acc_sh.at[pl.ds(s * INIT_ROWS, INIT_ROWS)])

        pltpu.sync_copy(dst_hbm.at[pl.ds(dst_row_base + w * groups, groups)],
                        idx_v)
        plsc.subcore_barrier()

        sems = (sem0, sem1)
        pltpu.async_copy(msg_hbm.at[pl.ds(base, CH)], buf_v.at[0], sems[0])

        def pair(k, carry):
            for b in range(2):
                g = 2 * k + b
                pltpu.make_async_copy(msg_hbm.at[pl.ds(base + g * CH, CH)],
                                      buf_v.at[b], sems[b]).wait()

                @pl.when(g + 1 < groups)
                def _next():
                    pltpu.async_copy(msg_hbm.at[pl.ds(base + (g + 1) * CH, CH)],
                                     buf_v.at[1 - b], sems[1 - b])

                pltpu.sync_copy(buf_v.at[b], acc_sh.at[idx_v.at[g]], add=True)
            return carry

        lax.fori_loop(0, groups // 2, pair, 0)
        plsc.subcore_barrier()

        @pl.when(s < INIT_TILES)
        def _out():
            pltpu.sync_copy(acc_sh.at[pl.ds(s * INIT_ROWS, INIT_ROWS)],
                            out_hbm.at[c, pl.ds(s * INIT_ROWS, INIT_ROWS)])

    return pl.kernel(
        body,
        mesh=plsc.VectorSubcoreMesh(core_axis_name="c", subcore_axis_name="s"),
        out_type=jax.ShapeDtypeStruct((NC, N_ATOMS, D), jnp.float32),
        scratch_types=[
            pltpu.VMEM_SHARED((ACC_ROWS, D), jnp.float32),
            pltpu.VMEM((groups, CH), jnp.int32),
            pltpu.VMEM((2, CH, D), jnp.float32),
            pltpu.SemaphoreType.DMA,
            pltpu.SemaphoreType.DMA,
        ],
    )


def _combine_body(p_ref, a_ref, o_ref):
    o_ref[...] = p_ref[0] + p_ref[1] - a_ref[...]


def _combine(partials, atom_features):
    return pl.pallas_call(
        _combine_body,
        grid=(N_ATOMS // COMBINE_BLOCK,),
        in_specs=[
            pl.BlockSpec((NC, COMBINE_BLOCK, D), lambda i: (0, i, 0)),
            pl.BlockSpec((COMBINE_BLOCK, D), lambda i: (i, 0)),
        ],
        out_specs=pl.BlockSpec((COMBINE_BLOCK, D), lambda i: (i, 0)),
        out_shape=jax.ShapeDtypeStruct((N_ATOMS, D), jnp.float32),
    )(partials, atom_features)


def kernel(atom_features, bond_features, bond_atom_indices, W_main, b_main, W_gate, b_gate):
    dst = bond_atom_indices[:, 1]
    dst_pad = jnp.concatenate([dst, jnp.asarray(_PAD_IDX)]).reshape(
        BONDS_PAD // CH, CH)
    msg0 = _mlp_chunk(0, CHUNK0_ROWS, bond_features,
                      W_main, b_main, W_gate, b_gate)
    msg1 = _mlp_chunk(CHUNK0_ROWS // MLP_BLOCK, CHUNK1_ROWS, bond_features,
                      W_main, b_main, W_gate, b_gate)
    p0 = _make_sc_scatter(True, GROUPS0, 0)(msg0, dst_pad, atom_features)
    p1 = _make_sc_scatter(False, GROUPS1, CHUNK0_ROWS // CH)(msg1, dst_pad, p0)
    return _combine(p1, atom_features)
